# baseline (speedup 1.0000x reference)
import functools

import jax
import jax.numpy as jnp
from jax import lax
from jax.experimental import pallas as pl
from jax.experimental.pallas import tpu as pltpu
from jax.experimental.pallas import tpu_sc as plsc

_D = 64
_DP = 128
_NC = 2
_NS = 16
_NW = _NC * _NS
_L = 16


@functools.lru_cache(maxsize=None)
def _gather_kernel(NB, NH, CB):
    b_per_w = NB // _NW
    nchunk = b_per_w // CB
    mesh = plsc.VectorSubcoreMesh(core_axis_name="c", subcore_axis_name="s")

    @functools.partial(
        pl.kernel,
        out_type=jax.ShapeDtypeStruct((NH, _D, NB), jnp.float32),
        mesh=mesh,
        scratch_types=[
            pltpu.VMEM((CB,), jnp.int32),
            pltpu.VMEM((CB, _DP), jnp.float32),
            pltpu.VMEM((_D, CB), jnp.float32),
            pltpu.SemaphoreType.DMA,
        ],
        compiler_params=pltpu.CompilerParams(
            use_tc_tiling_on_sc=True, needs_layout_passes=False
        ),
    )
    def k(tab_hbm, xt_hbm, out_hbm, idx_v, rows_v, t_v, sem):
        wid = lax.axis_index("s") * _NC + lax.axis_index("c")
        b0w = wid * b_per_w
        lanes = lax.iota(jnp.int32, _L)
        row_ids = [lanes + k16 * _L for k16 in range(_D // _L)]

        @pl.loop(0, NH)
        def _(h):
            @pl.loop(0, nchunk)
            def _(c):
                b0 = b0w + c * CB
                pltpu.sync_copy(xt_hbm.at[h, pl.ds(b0, CB)], idx_v)
                pltpu.async_copy(tab_hbm.at[idx_v], rows_v, sem).wait()

                @pl.loop(0, CB)
                def _(b):
                    col = jnp.full((_L,), b, jnp.int32)
                    for k16 in range(_D // _L):
                        v = rows_v.at[b][pl.ds(k16 * _L, _L)]
                        plsc.store_scatter(t_v, [row_ids[k16], col], v)

                pltpu.sync_copy(t_v, out_hbm.at[h, :, pl.ds(b0, CB)])

    return k


def kernel(x, lut_weight):
    nb, nh = x.shape
    xt = x.T
    tab = jnp.pad(lut_weight, ((0, 0), (0, _DP - _D)))
    out3 = _gather_kernel(nb, nh, 256)(tab, xt)
    return out3.transpose(2, 0, 1)


# speedup vs baseline: 1.0508x; 1.0508x over previous
import functools

import jax
import jax.numpy as jnp
from jax import lax
from jax.experimental import pallas as pl
from jax.experimental.pallas import tpu as pltpu
from jax.experimental.pallas import tpu_sc as plsc

_D = 64
_DP = 128
_NC = 2
_NS = 16
_NW = _NC * _NS
_L = 16


@functools.lru_cache(maxsize=None)
def _gather_kernel(NB, NH, CB):
    b_per_w = NB // _NW
    nchunk = b_per_w // CB
    assert nchunk == 2
    ntask = NH * nchunk
    mesh = plsc.VectorSubcoreMesh(core_axis_name="c", subcore_axis_name="s")

    @functools.partial(
        pl.kernel,
        out_type=jax.ShapeDtypeStruct((NH, _D, NB), jnp.float32),
        mesh=mesh,
        scratch_types=[
            pltpu.VMEM((NH * b_per_w,), jnp.int32),
            pltpu.VMEM((CB, _DP), jnp.float32),
            pltpu.VMEM((CB, _DP), jnp.float32),
            pltpu.VMEM((_D, CB), jnp.float32),
            pltpu.VMEM((_D, CB), jnp.float32),
            pltpu.SemaphoreType.DMA,
            pltpu.SemaphoreType.DMA,
            pltpu.SemaphoreType.DMA,
            pltpu.SemaphoreType.DMA,
        ],
        compiler_params=pltpu.CompilerParams(
            use_tc_tiling_on_sc=True, needs_layout_passes=False
        ),
    )
    def k(tab_hbm, xt_hbm, out_hbm, idx_v, r0, r1, t0, t1, g0, g1, s0, s1):
        wid = lax.axis_index("s") * _NC + lax.axis_index("c")
        b0w = wid * b_per_w
        rows = (r0, r1)
        tbuf = (t0, t1)
        gsem = (g0, g1)
        ssem = (s0, s1)
        lanes = lax.iota(jnp.int32, _L)
        row_sel = [c * _L + lanes for c in range(CB // _L)]

        for h in range(NH):
            pltpu.async_copy(
                xt_hbm.at[h, pl.ds(b0w, b_per_w)],
                idx_v.at[pl.ds(h * b_per_w, b_per_w)],
                g0,
            )
        for h in range(NH):
            pltpu.make_async_copy(
                xt_hbm.at[h, pl.ds(b0w, b_per_w)],
                idx_v.at[pl.ds(h * b_per_w, b_per_w)],
                g0,
            ).wait()
        for b in range(2):
            pltpu.async_copy(
                tab_hbm.at[idx_v.at[pl.ds(b * CB, CB)]], rows[b], gsem[b]
            )

        @pl.loop(0, ntask // 2)
        def _(i):
            for b in range(2):
                t = 2 * i + b

                @pl.when(t >= 2)
                def _():
                    pltpu.make_async_copy(
                        tbuf[b],
                        out_hbm.at[i - 1, :, pl.ds(b0w + b * CB, CB)],
                        ssem[b],
                    ).wait()

                pltpu.make_async_copy(
                    tab_hbm.at[pl.ds(0, CB)], rows[b], gsem[b]
                ).wait()

                @pl.loop(0, _D)
                def _(d):
                    dcol = jnp.full((_L,), d, jnp.int32)
                    for c in range(CB // _L):
                        v = plsc.load_gather(rows[b], [row_sel[c], dcol])
                        plsc.store_scatter(tbuf[b], [dcol, row_sel[c]], v)

                pltpu.async_copy(
                    tbuf[b],
                    out_hbm.at[i, :, pl.ds(b0w + b * CB, CB)],
                    ssem[b],
                )

                @pl.when(t + 2 < ntask)
                def _():
                    pltpu.async_copy(
                        tab_hbm.at[
                            idx_v.at[pl.ds((i + 1) * b_per_w + b * CB, CB)]
                        ],
                        rows[b],
                        gsem[b],
                    )

        for b in range(2):
            pltpu.make_async_copy(
                tbuf[b],
                out_hbm.at[NH - 1, :, pl.ds(b0w + b * CB, CB)],
                ssem[b],
            ).wait()

    return k


def kernel(x, lut_weight):
    nb, nh = x.shape
    xt = x.T
    tab = jnp.pad(lut_weight, ((0, 0), (0, _DP - _D)))
    out3 = _gather_kernel(nb, nh, 256)(tab, xt)
    return out3.transpose(2, 0, 1)


# speedup vs baseline: 1.6803x; 1.5990x over previous
import functools

import jax
import jax.numpy as jnp
from jax import lax
from jax.experimental import pallas as pl
from jax.experimental.pallas import tpu as pltpu
from jax.experimental.pallas import tpu_sc as plsc

_D = 64
_DP = 128
_NC = 2
_NS = 16
_NW = _NC * _NS
_L = 16


@functools.lru_cache(maxsize=None)
def _gather_kernel(NB, NH, CB):
    b_per_w = NB // _NW
    nchunk = b_per_w // CB
    assert nchunk == 2
    ntask = NH * nchunk
    mesh = plsc.VectorSubcoreMesh(core_axis_name="c", subcore_axis_name="s")

    @functools.partial(
        pl.kernel,
        out_type=jax.ShapeDtypeStruct((NH, _D, NB), jnp.float32),
        mesh=mesh,
        scratch_types=[
            pltpu.VMEM((NH * b_per_w,), jnp.int32),
            pltpu.VMEM((CB, _DP), jnp.float32),
            pltpu.VMEM((CB, _DP), jnp.float32),
            pltpu.VMEM((_D, CB), jnp.float32),
            pltpu.VMEM((_D, CB), jnp.float32),
            pltpu.SemaphoreType.DMA,
            pltpu.SemaphoreType.DMA,
            pltpu.SemaphoreType.DMA,
            pltpu.SemaphoreType.DMA,
        ],
        compiler_params=pltpu.CompilerParams(
            use_tc_tiling_on_sc=True, needs_layout_passes=False
        ),
    )
    def k(tab_hbm, xt_hbm, out_hbm, idx_v, r0, r1, t0, t1, g0, g1, s0, s1):
        wid = lax.axis_index("s") * _NC + lax.axis_index("c")
        b0w = wid * b_per_w
        rows = (r0, r1)
        tbuf = (t0, t1)
        gsem = (g0, g1)
        ssem = (s0, s1)
        lanes = lax.iota(jnp.int32, _L)
        rot = [(lanes + j) & (_L - 1) for j in range(_L)]

        for h in range(NH):
            pltpu.async_copy(
                xt_hbm.at[h, pl.ds(b0w, b_per_w)],
                idx_v.at[pl.ds(h * b_per_w, b_per_w)],
                g0,
            )
        for h in range(NH):
            pltpu.make_async_copy(
                xt_hbm.at[h, pl.ds(b0w, b_per_w)],
                idx_v.at[pl.ds(h * b_per_w, b_per_w)],
                g0,
            ).wait()
        for b in range(2):
            pltpu.async_copy(
                tab_hbm.at[idx_v.at[pl.ds(b * CB, CB)]], rows[b], gsem[b]
            )

        @pl.loop(0, ntask // 2)
        def _(i):
            for b in range(2):
                t = 2 * i + b

                @pl.when(t >= 2)
                def _():
                    pltpu.make_async_copy(
                        tbuf[b],
                        out_hbm.at[i - 1, :, pl.ds(b0w + b * CB, CB)],
                        ssem[b],
                    ).wait()

                pltpu.make_async_copy(
                    tab_hbm.at[pl.ds(0, CB)], rows[b], gsem[b]
                ).wait()

                @pl.loop(0, CB // _L)
                def _(c):
                    bvec = jnp.full((_L,), c * _L, jnp.int32) + lanes
                    for g in range(_D // _L):
                        for j in range(_L):
                            dvec = rot[j] + g * _L
                            v = plsc.load_gather(rows[b], [bvec, dvec])
                            plsc.store_scatter(tbuf[b], [dvec, bvec], v)

                pltpu.async_copy(
                    tbuf[b],
                    out_hbm.at[i, :, pl.ds(b0w + b * CB, CB)],
                    ssem[b],
                )

                @pl.when(t + 2 < ntask)
                def _():
                    pltpu.async_copy(
                        tab_hbm.at[
                            idx_v.at[pl.ds((i + 1) * b_per_w + b * CB, CB)]
                        ],
                        rows[b],
                        gsem[b],
                    )

        for b in range(2):
            pltpu.make_async_copy(
                tbuf[b],
                out_hbm.at[NH - 1, :, pl.ds(b0w + b * CB, CB)],
                ssem[b],
            ).wait()

    return k


def kernel(x, lut_weight):
    nb, nh = x.shape
    xt = x.T
    tab = jnp.pad(lut_weight, ((0, 0), (0, _DP - _D)))
    out3 = _gather_kernel(nb, nh, 256)(tab, xt)
    return out3.transpose(2, 0, 1)


# speedup vs baseline: 1.7212x; 1.0244x over previous
import functools

import jax
import jax.numpy as jnp
from jax import lax
from jax.experimental import pallas as pl
from jax.experimental.pallas import tpu as pltpu
from jax.experimental.pallas import tpu_sc as plsc

_D = 64
_DP = 128
_NC = 2
_NS = 16
_NW = _NC * _NS
_L = 16


@functools.lru_cache(maxsize=None)
def _gather_kernel(NB, NH, CB):
    b_per_w = NB // _NW
    nchunk = b_per_w // CB
    assert nchunk == 2
    ntask = NH * nchunk
    mesh = plsc.VectorSubcoreMesh(core_axis_name="c", subcore_axis_name="s")

    @functools.partial(
        pl.kernel,
        out_type=jax.ShapeDtypeStruct((NH, _D, NB), jnp.float32),
        mesh=mesh,
        scratch_types=[
            pltpu.VMEM((NH * b_per_w,), jnp.int32),
            pltpu.VMEM((CB, _DP), jnp.float32),
            pltpu.VMEM((CB, _DP), jnp.float32),
            pltpu.VMEM((_D, CB), jnp.float32),
            pltpu.VMEM((_D, CB), jnp.float32),
            pltpu.SemaphoreType.DMA,
            pltpu.SemaphoreType.DMA,
            pltpu.SemaphoreType.DMA,
            pltpu.SemaphoreType.DMA,
        ],
        compiler_params=pltpu.CompilerParams(
            use_tc_tiling_on_sc=True, needs_layout_passes=False
        ),
    )
    def k(tab_hbm, xt_hbm, out_hbm, idx_v, r0, r1, t0, t1, g0, g1, s0, s1):
        wid = lax.axis_index("s") * _NC + lax.axis_index("c")
        b0w = wid * b_per_w
        rows = (r0, r1)
        tbuf = (t0, t1)
        gsem = (g0, g1)
        ssem = (s0, s1)
        lanes = lax.iota(jnp.int32, _L)
        rot = [(lanes + j) & (_L - 1) for j in range(_L)]

        for h in range(NH):
            pltpu.async_copy(
                xt_hbm.at[h, pl.ds(b0w, b_per_w)],
                idx_v.at[pl.ds(h * b_per_w, b_per_w)],
                g0,
            )
        for h in range(NH):
            pltpu.make_async_copy(
                xt_hbm.at[h, pl.ds(b0w, b_per_w)],
                idx_v.at[pl.ds(h * b_per_w, b_per_w)],
                g0,
            ).wait()
        for b in range(2):
            pltpu.async_copy(
                tab_hbm.at[idx_v.at[pl.ds(b * CB, CB)]], rows[b], gsem[b]
            )

        @pl.loop(0, ntask // 2)
        def _(i):
            for b in range(2):
                t = 2 * i + b

                @pl.when(t >= 2)
                def _():
                    pltpu.make_async_copy(
                        tbuf[b],
                        out_hbm.at[i - 1, :, pl.ds(b0w + b * CB, CB)],
                        ssem[b],
                    ).wait()

                pltpu.make_async_copy(
                    tab_hbm.at[pl.ds(0, CB)], rows[b], gsem[b]
                ).wait()

                @pl.loop(0, CB // _L, unroll=2)
                def _(c):
                    bvec = jnp.full((_L,), c * _L, jnp.int32) + lanes
                    for g in range(_D // _L):
                        for j in range(_L):
                            dvec = rot[j] + g * _L
                            v = plsc.load_gather(rows[b], [bvec, dvec])
                            plsc.store_scatter(tbuf[b], [dvec, bvec], v)

                pltpu.async_copy(
                    tbuf[b],
                    out_hbm.at[i, :, pl.ds(b0w + b * CB, CB)],
                    ssem[b],
                )

                @pl.when(t + 2 < ntask)
                def _():
                    pltpu.async_copy(
                        tab_hbm.at[
                            idx_v.at[pl.ds((i + 1) * b_per_w + b * CB, CB)]
                        ],
                        rows[b],
                        gsem[b],
                    )

        for b in range(2):
            pltpu.make_async_copy(
                tbuf[b],
                out_hbm.at[NH - 1, :, pl.ds(b0w + b * CB, CB)],
                ssem[b],
            ).wait()

    return k


def kernel(x, lut_weight):
    nb, nh = x.shape
    xt = x.T
    tab = jnp.pad(lut_weight, ((0, 0), (0, _DP - _D)))
    out3 = _gather_kernel(nb, nh, 256)(tab, xt)
    return out3.transpose(2, 0, 1)


# speedup vs baseline: 1.8745x; 1.0891x over previous
import functools

import jax
import jax.numpy as jnp
from jax import lax
from jax.experimental import pallas as pl
from jax.experimental.pallas import tpu as pltpu
from jax.experimental.pallas import tpu_sc as plsc

_D = 64
_DP = 128
_NC = 2
_NS = 16
_NW = _NC * _NS
_L = 16


@functools.lru_cache(maxsize=None)
def _format_kernel(V, VC):
    nchunks = V // VC
    tail = V - nchunks * VC
    base_per_w = nchunks // _NW
    extra = nchunks % _NW
    niter = base_per_w + (1 if extra else 0)
    npair = (niter + 1) // 2
    assert VC % _DP == 0 and tail % 8 == 0
    mesh = plsc.VectorSubcoreMesh(core_axis_name="c", subcore_axis_name="s")

    @functools.partial(
        pl.kernel,
        out_type=jax.ShapeDtypeStruct((V, _DP), jnp.float32),
        mesh=mesh,
        scratch_types=[
            pltpu.VMEM((_D, VC), jnp.float32),
            pltpu.VMEM((_D, VC), jnp.float32),
            pltpu.VMEM((VC, _DP), jnp.float32),
            pltpu.VMEM((VC, _DP), jnp.float32),
            pltpu.VMEM((_D, _DP), jnp.float32),
            pltpu.SemaphoreType.DMA,
            pltpu.SemaphoreType.DMA,
            pltpu.SemaphoreType.DMA,
            pltpu.SemaphoreType.DMA,
        ],
        compiler_params=pltpu.CompilerParams(
            use_tc_tiling_on_sc=True, needs_layout_passes=False
        ),
    )
    def k(lut_hbm, tl_hbm, tab_hbm, i0, i1, o0, o1, tl_v, gi0, gi1, so0, so1):
        wid = lax.axis_index("s") * _NC + lax.axis_index("c")
        inb = (i0, i1)
        ob = (o0, o1)
        gsem = (gi0, gi1)
        ssem = (so0, so1)
        lanes = lax.iota(jnp.int32, _L)
        rot = [(lanes + j) & (_L - 1) for j in range(_L)]
        dsel = [g * _L + lanes for g in range(_D // _L)]
        nmine = base_per_w + jnp.where(wid < extra, 1, 0)

        def v0_of(kk):
            return (kk * _NW + wid) * VC

        for kk in range(2):
            @pl.when(kk < nmine)
            def _():
                pltpu.async_copy(
                    lut_hbm.at[:, pl.ds(v0_of(kk), VC)], inb[kk], gsem[kk]
                )

        @pl.loop(0, npair)
        def _(i):
            for b in range(2):
                kk = 2 * i + b

                @pl.when(kk < nmine)
                def _():
                    @pl.when(kk >= 2)
                    def _():
                        pltpu.make_async_copy(
                            ob[b],
                            tab_hbm.at[pl.ds(v0_of(kk - 2), VC)],
                            ssem[b],
                        ).wait()

                    pltpu.make_async_copy(
                        lut_hbm.at[:, pl.ds(0, VC)], inb[b], gsem[b]
                    ).wait()

                    @pl.loop(0, VC // _L, unroll=2)
                    def _(vb):
                        vbase = jnp.full((_L,), vb * _L, jnp.int32)
                        for g in range(_D // _L):
                            for j in range(_L):
                                vvec = vbase + rot[j]
                                v = plsc.load_gather(inb[b], [dsel[g], vvec])
                                plsc.store_scatter(ob[b], [vvec, dsel[g]], v)

                    pltpu.async_copy(
                        ob[b], tab_hbm.at[pl.ds(v0_of(kk), VC)], ssem[b]
                    )

                    @pl.when(kk + 2 < nmine)
                    def _():
                        pltpu.async_copy(
                            lut_hbm.at[:, pl.ds(v0_of(kk + 2), VC)],
                            inb[b],
                            gsem[b],
                        )

        for b in range(2):
            last = nmine - 1 - ((nmine - 1 - b) & 1)

            @pl.when(last >= 0)
            def _():
                pltpu.make_async_copy(
                    ob[b],
                    tab_hbm.at[pl.ds(v0_of(last), VC)],
                    ssem[b],
                ).wait()

        if tail:
            @pl.when(wid == _NW - 1)
            def _():
                pltpu.sync_copy(tl_hbm, tl_v)
                pltpu.sync_copy(
                    tl_v.at[pl.ds(0, tail), :],
                    tab_hbm.at[pl.ds(nchunks * VC, tail)],
                )

    return k


@functools.lru_cache(maxsize=None)
def _gather_kernel(NB, NH, CB):
    b_per_w = NB // _NW
    nchunk = b_per_w // CB
    assert nchunk == 2
    ntask = NH * nchunk
    mesh = plsc.VectorSubcoreMesh(core_axis_name="c", subcore_axis_name="s")

    @functools.partial(
        pl.kernel,
        out_type=jax.ShapeDtypeStruct((NH, _D, NB), jnp.float32),
        mesh=mesh,
        scratch_types=[
            pltpu.VMEM((NH * b_per_w,), jnp.int32),
            pltpu.VMEM((CB, _DP), jnp.float32),
            pltpu.VMEM((CB, _DP), jnp.float32),
            pltpu.VMEM((_D, CB), jnp.float32),
            pltpu.VMEM((_D, CB), jnp.float32),
            pltpu.SemaphoreType.DMA,
            pltpu.SemaphoreType.DMA,
            pltpu.SemaphoreType.DMA,
            pltpu.SemaphoreType.DMA,
        ],
        compiler_params=pltpu.CompilerParams(
            use_tc_tiling_on_sc=True, needs_layout_passes=False
        ),
    )
    def k(tab_hbm, xt_hbm, out_hbm, idx_v, r0, r1, t0, t1, g0, g1, s0, s1):
        wid = lax.axis_index("s") * _NC + lax.axis_index("c")
        b0w = wid * b_per_w
        rows = (r0, r1)
        tbuf = (t0, t1)
        gsem = (g0, g1)
        ssem = (s0, s1)
        lanes = lax.iota(jnp.int32, _L)
        rot = [(lanes + j) & (_L - 1) for j in range(_L)]

        for h in range(NH):
            pltpu.async_copy(
                xt_hbm.at[h, pl.ds(b0w, b_per_w)],
                idx_v.at[pl.ds(h * b_per_w, b_per_w)],
                g0,
            )
        for h in range(NH):
            pltpu.make_async_copy(
                xt_hbm.at[h, pl.ds(b0w, b_per_w)],
                idx_v.at[pl.ds(h * b_per_w, b_per_w)],
                g0,
            ).wait()
        for b in range(2):
            pltpu.async_copy(
                tab_hbm.at[idx_v.at[pl.ds(b * CB, CB)]], rows[b], gsem[b]
            )

        @pl.loop(0, ntask // 2)
        def _(i):
            for b in range(2):
                t = 2 * i + b

                @pl.when(t >= 2)
                def _():
                    pltpu.make_async_copy(
                        tbuf[b],
                        out_hbm.at[i - 1, :, pl.ds(b0w + b * CB, CB)],
                        ssem[b],
                    ).wait()

                pltpu.make_async_copy(
                    tab_hbm.at[pl.ds(0, CB)], rows[b], gsem[b]
                ).wait()

                @pl.loop(0, CB // _L, unroll=2)
                def _(c):
                    bvec = jnp.full((_L,), c * _L, jnp.int32) + lanes
                    for g in range(_D // _L):
                        for j in range(_L):
                            dvec = rot[j] + g * _L
                            v = plsc.load_gather(rows[b], [bvec, dvec])
                            plsc.store_scatter(tbuf[b], [dvec, bvec], v)

                pltpu.async_copy(
                    tbuf[b],
                    out_hbm.at[i, :, pl.ds(b0w + b * CB, CB)],
                    ssem[b],
                )

                @pl.when(t + 2 < ntask)
                def _():
                    pltpu.async_copy(
                        tab_hbm.at[
                            idx_v.at[pl.ds((i + 1) * b_per_w + b * CB, CB)]
                        ],
                        rows[b],
                        gsem[b],
                    )

        for b in range(2):
            pltpu.make_async_copy(
                tbuf[b],
                out_hbm.at[NH - 1, :, pl.ds(b0w + b * CB, CB)],
                ssem[b],
            ).wait()

    return k


def kernel(x, lut_weight):
    nb, nh = x.shape
    nv = lut_weight.shape[0]
    ntail = nv - nv // (2 * _DP) * (2 * _DP)
    tail_pad = jnp.pad(
        lut_weight[nv - ntail:], ((0, _D - ntail), (0, _DP - _D))
    )
    tab = _format_kernel(nv, 256)(lut_weight.T, tail_pad)
    out3 = _gather_kernel(nb, nh, 256)(tab, x.T)
    return out3.transpose(2, 0, 1)


# speedup vs baseline: 1.8932x; 1.0100x over previous
import functools

import jax
import jax.numpy as jnp
from jax import lax
from jax.experimental import pallas as pl
from jax.experimental.pallas import tpu as pltpu
from jax.experimental.pallas import tpu_sc as plsc

_D = 64
_DP = 128
_NC = 2
_NS = 16
_NW = _NC * _NS
_L = 16


@functools.lru_cache(maxsize=None)
def _format_kernel(V, VC):
    nchunks = V // VC
    tail = V - nchunks * VC
    base_per_w = nchunks // _NW
    extra = nchunks % _NW
    niter = base_per_w + (1 if extra else 0)
    npair = (niter + 1) // 2
    assert VC % _DP == 0 and tail % 8 == 0
    mesh = plsc.VectorSubcoreMesh(core_axis_name="c", subcore_axis_name="s")

    @functools.partial(
        pl.kernel,
        out_type=jax.ShapeDtypeStruct((V // 2, _DP), jnp.float32),
        mesh=mesh,
        scratch_types=[
            pltpu.VMEM((_D, VC), jnp.float32),
            pltpu.VMEM((_D, VC), jnp.float32),
            pltpu.VMEM((VC // 2, _DP), jnp.float32),
            pltpu.VMEM((VC // 2, _DP), jnp.float32),
            pltpu.VMEM((32, _DP), jnp.float32),
            pltpu.SemaphoreType.DMA,
            pltpu.SemaphoreType.DMA,
            pltpu.SemaphoreType.DMA,
            pltpu.SemaphoreType.DMA,
        ],
        compiler_params=pltpu.CompilerParams(
            use_tc_tiling_on_sc=True, needs_layout_passes=False
        ),
    )
    def k(lut_hbm, tl_hbm, tab_hbm, i0, i1, o0, o1, tl_v, gi0, gi1, so0, so1):
        wid = lax.axis_index("s") * _NC + lax.axis_index("c")
        inb = (i0, i1)
        ob = (o0, o1)
        gsem = (gi0, gi1)
        ssem = (so0, so1)
        lanes = lax.iota(jnp.int32, _L)
        rot = [(lanes + j) & (_L - 1) for j in range(_L)]
        roth = [lax.shift_right_logical(r, 1) for r in rot]
        rpar = [lax.shift_left(r & 1, 6) for r in rot]
        dsel = [g * _L + lanes for g in range(_D // _L)]
        nmine = base_per_w + jnp.where(wid < extra, 1, 0)

        def v0_of(kk):
            return (kk * _NW + wid) * VC

        def o0_of(kk):
            return (kk * _NW + wid) * (VC // 2)

        for kk in range(2):
            @pl.when(kk < nmine)
            def _():
                pltpu.async_copy(
                    lut_hbm.at[:, pl.ds(v0_of(kk), VC)], inb[kk], gsem[kk]
                )

        @pl.loop(0, npair)
        def _(i):
            for b in range(2):
                kk = 2 * i + b

                @pl.when(kk < nmine)
                def _():
                    @pl.when(kk >= 2)
                    def _():
                        pltpu.make_async_copy(
                            ob[b],
                            tab_hbm.at[pl.ds(o0_of(kk - 2), VC // 2)],
                            ssem[b],
                        ).wait()

                    pltpu.make_async_copy(
                        lut_hbm.at[:, pl.ds(0, VC)], inb[b], gsem[b]
                    ).wait()

                    @pl.loop(0, VC // _L, unroll=2)
                    def _(vb):
                        vbase = jnp.full((_L,), vb * _L, jnp.int32)
                        vbase8 = jnp.full((_L,), vb * (_L // 2), jnp.int32)
                        for g in range(_D // _L):
                            for j in range(_L):
                                vvec = vbase + rot[j]
                                v = plsc.load_gather(inb[b], [dsel[g], vvec])
                                plsc.store_scatter(
                                    ob[b],
                                    [vbase8 + roth[j], dsel[g] + rpar[j]],
                                    v,
                                )

                    pltpu.async_copy(
                        ob[b],
                        tab_hbm.at[pl.ds(o0_of(kk), VC // 2)],
                        ssem[b],
                    )

                    @pl.when(kk + 2 < nmine)
                    def _():
                        pltpu.async_copy(
                            lut_hbm.at[:, pl.ds(v0_of(kk + 2), VC)],
                            inb[b],
                            gsem[b],
                        )

        for b in range(2):
            last = nmine - 1 - ((nmine - 1 - b) & 1)

            @pl.when(last >= 0)
            def _():
                pltpu.make_async_copy(
                    ob[b],
                    tab_hbm.at[pl.ds(o0_of(last), VC // 2)],
                    ssem[b],
                ).wait()

        if tail:
            @pl.when(wid == _NW - 1)
            def _():
                pltpu.sync_copy(tl_hbm, tl_v)
                pltpu.sync_copy(
                    tl_v.at[pl.ds(0, tail // 2), :],
                    tab_hbm.at[pl.ds(nchunks * VC // 2, tail // 2)],
                )

    return k


@functools.lru_cache(maxsize=None)
def _gather_kernel(NB, NH, CB):
    b_per_w = NB // _NW
    nchunk = b_per_w // CB
    assert nchunk == 2
    ntask = NH * nchunk
    mesh = plsc.VectorSubcoreMesh(core_axis_name="c", subcore_axis_name="s")

    @functools.partial(
        pl.kernel,
        out_type=jax.ShapeDtypeStruct((NH, _D, NB), jnp.float32),
        mesh=mesh,
        scratch_types=[
            pltpu.VMEM((NH * b_per_w,), jnp.int32),
            pltpu.VMEM((CB,), jnp.int32),
            pltpu.VMEM((CB,), jnp.int32),
            pltpu.VMEM((CB, _DP), jnp.float32),
            pltpu.VMEM((CB, _DP), jnp.float32),
            pltpu.VMEM((_D, CB), jnp.float32),
            pltpu.VMEM((_D, CB), jnp.float32),
            pltpu.SemaphoreType.DMA,
            pltpu.SemaphoreType.DMA,
            pltpu.SemaphoreType.DMA,
            pltpu.SemaphoreType.DMA,
        ],
        compiler_params=pltpu.CompilerParams(
            use_tc_tiling_on_sc=True, needs_layout_passes=False
        ),
    )
    def k(tab_hbm, xt_hbm, out_hbm, idx_v, ig0, ig1, r0, r1,
          t0, t1, g0, g1, s0, s1):
        wid = lax.axis_index("s") * _NC + lax.axis_index("c")
        b0w = wid * b_per_w
        rows = (r0, r1)
        idxg = (ig0, ig1)
        tbuf = (t0, t1)
        gsem = (g0, g1)
        ssem = (s0, s1)
        lanes = lax.iota(jnp.int32, _L)
        rot = [(lanes + j) & (_L - 1) for j in range(_L)]

        for h in range(NH):
            pltpu.async_copy(
                xt_hbm.at[h, pl.ds(b0w, b_per_w)],
                idx_v.at[pl.ds(h * b_per_w, b_per_w)],
                g0,
            )
        for h in range(NH):
            pltpu.make_async_copy(
                xt_hbm.at[h, pl.ds(b0w, b_per_w)],
                idx_v.at[pl.ds(h * b_per_w, b_per_w)],
                g0,
            ).wait()
        def fill_idxg(b, off):
            for c in range(CB // _L):
                iv = idx_v[pl.ds(off + c * _L, _L)]
                idxg[b][pl.ds(c * _L, _L)] = lax.shift_right_logical(iv, 1)

        for b in range(2):
            fill_idxg(b, b * CB)
            pltpu.async_copy(tab_hbm.at[idxg[b]], rows[b], gsem[b])

        @pl.loop(0, ntask // 2)
        def _(i):
            for b in range(2):
                t = 2 * i + b

                @pl.when(t >= 2)
                def _():
                    pltpu.make_async_copy(
                        tbuf[b],
                        out_hbm.at[i - 1, :, pl.ds(b0w + b * CB, CB)],
                        ssem[b],
                    ).wait()

                pltpu.make_async_copy(
                    tab_hbm.at[pl.ds(0, CB)], rows[b], gsem[b]
                ).wait()

                toff = i * b_per_w + b * CB

                @pl.loop(0, CB // _L, unroll=2)
                def _(c):
                    bvec = jnp.full((_L,), c * _L, jnp.int32) + lanes
                    iv = idx_v[pl.ds(toff + c * _L, _L)]
                    par = lax.shift_left(iv & 1, 6)
                    for g in range(_D // _L):
                        for j in range(_L):
                            dvec = rot[j] + g * _L
                            v = plsc.load_gather(rows[b], [bvec, dvec + par])
                            plsc.store_scatter(tbuf[b], [dvec, bvec], v)

                pltpu.async_copy(
                    tbuf[b],
                    out_hbm.at[i, :, pl.ds(b0w + b * CB, CB)],
                    ssem[b],
                )

                @pl.when(t + 2 < ntask)
                def _():
                    fill_idxg(b, (i + 1) * b_per_w + b * CB)
                    pltpu.async_copy(tab_hbm.at[idxg[b]], rows[b], gsem[b])

        for b in range(2):
            pltpu.make_async_copy(
                tbuf[b],
                out_hbm.at[NH - 1, :, pl.ds(b0w + b * CB, CB)],
                ssem[b],
            ).wait()

    return k


def kernel(x, lut_weight):
    nb, nh = x.shape
    nv = lut_weight.shape[0]
    nfull = nv // _DP * _DP
    tl2 = lut_weight[nfull:].reshape((nv - nfull) // 2, _DP)
    tab = _format_kernel(nv, 384)(lut_weight.T, tl2)
    out3 = _gather_kernel(nb, nh, 256)(tab, x.T)
    return out3.transpose(2, 0, 1)


# speedup vs baseline: 1.9516x; 1.0308x over previous
import functools

import jax
import jax.numpy as jnp
from jax import lax
from jax.experimental import pallas as pl
from jax.experimental.pallas import tpu as pltpu
from jax.experimental.pallas import tpu_sc as plsc

_D = 64
_DP = 128
_NC = 2
_NS = 16
_NW = _NC * _NS
_L = 16


@functools.lru_cache(maxsize=None)
def _format_kernel(V, VC):
    nchunks = V // VC
    tail = V - nchunks * VC
    base_per_w = nchunks // _NW
    extra = nchunks % _NW
    niter = base_per_w + (1 if extra else 0)
    npair = (niter + 1) // 2
    assert VC % _DP == 0 and tail % 8 == 0
    mesh = plsc.VectorSubcoreMesh(core_axis_name="c", subcore_axis_name="s")

    @functools.partial(
        pl.kernel,
        out_type=jax.ShapeDtypeStruct((V // 2, _DP), jnp.float32),
        mesh=mesh,
        scratch_types=[
            pltpu.VMEM((_D, VC), jnp.float32),
            pltpu.VMEM((_D, VC), jnp.float32),
            pltpu.VMEM((VC // 2, _DP), jnp.float32),
            pltpu.VMEM((VC // 2, _DP), jnp.float32),
            pltpu.VMEM((32, _DP), jnp.float32),
            pltpu.SemaphoreType.DMA,
            pltpu.SemaphoreType.DMA,
            pltpu.SemaphoreType.DMA,
            pltpu.SemaphoreType.DMA,
        ],
        compiler_params=pltpu.CompilerParams(
            use_tc_tiling_on_sc=True, needs_layout_passes=False
        ),
    )
    def k(lut_hbm, tl_hbm, tab_hbm, i0, i1, o0, o1, tl_v, gi0, gi1, so0, so1):
        wid = lax.axis_index("s") * _NC + lax.axis_index("c")
        inb = (i0, i1)
        ob = (o0, o1)
        gsem = (gi0, gi1)
        ssem = (so0, so1)
        lanes = lax.iota(jnp.int32, _L)
        rot = [(lanes + j) & (_L - 1) for j in range(_L)]
        roth = [lax.shift_right_logical(r, 1) for r in rot]
        rpar = [lax.shift_left(r & 1, 6) for r in rot]
        dsel = [g * _L + lanes for g in range(_D // _L)]
        nmine = base_per_w + jnp.where(wid < extra, 1, 0)

        def v0_of(kk):
            return (kk * _NW + wid) * VC

        def o0_of(kk):
            return (kk * _NW + wid) * (VC // 2)

        for kk in range(2):
            @pl.when(kk < nmine)
            def _():
                pltpu.async_copy(
                    lut_hbm.at[:, pl.ds(v0_of(kk), VC)], inb[kk], gsem[kk]
                )

        @pl.loop(0, npair)
        def _(i):
            for b in range(2):
                kk = 2 * i + b

                @pl.when(kk < nmine)
                def _():
                    @pl.when(kk >= 2)
                    def _():
                        pltpu.make_async_copy(
                            ob[b],
                            tab_hbm.at[pl.ds(o0_of(kk - 2), VC // 2)],
                            ssem[b],
                        ).wait()

                    pltpu.make_async_copy(
                        lut_hbm.at[:, pl.ds(0, VC)], inb[b], gsem[b]
                    ).wait()

                    @pl.loop(0, VC // _L, unroll=4)
                    def _(vb):
                        vbase = jnp.full((_L,), vb * _L, jnp.int32)
                        vbase8 = jnp.full((_L,), vb * (_L // 2), jnp.int32)
                        for g in range(_D // _L):
                            for j in range(_L):
                                vvec = vbase + rot[j]
                                v = plsc.load_gather(inb[b], [dsel[g], vvec])
                                plsc.store_scatter(
                                    ob[b],
                                    [vbase8 + roth[j], dsel[g] + rpar[j]],
                                    v,
                                )

                    pltpu.async_copy(
                        ob[b],
                        tab_hbm.at[pl.ds(o0_of(kk), VC // 2)],
                        ssem[b],
                    )

                    @pl.when(kk + 2 < nmine)
                    def _():
                        pltpu.async_copy(
                            lut_hbm.at[:, pl.ds(v0_of(kk + 2), VC)],
                            inb[b],
                            gsem[b],
                        )

        for b in range(2):
            last = nmine - 1 - ((nmine - 1 - b) & 1)

            @pl.when(last >= 0)
            def _():
                pltpu.make_async_copy(
                    ob[b],
                    tab_hbm.at[pl.ds(o0_of(last), VC // 2)],
                    ssem[b],
                ).wait()

        if tail:
            @pl.when(wid == _NW - 1)
            def _():
                pltpu.sync_copy(tl_hbm, tl_v)
                pltpu.sync_copy(
                    tl_v.at[pl.ds(0, tail // 2), :],
                    tab_hbm.at[pl.ds(nchunks * VC // 2, tail // 2)],
                )

    return k


@functools.lru_cache(maxsize=None)
def _gather_kernel(NB, NH, CB):
    b_per_w = NB // _NW
    nchunk = b_per_w // CB
    assert nchunk == 2
    ntask = NH * nchunk
    mesh = plsc.VectorSubcoreMesh(core_axis_name="c", subcore_axis_name="s")

    @functools.partial(
        pl.kernel,
        out_type=jax.ShapeDtypeStruct((NH, _D, NB), jnp.float32),
        mesh=mesh,
        scratch_types=[
            pltpu.VMEM((NH * b_per_w,), jnp.int32),
            pltpu.VMEM((CB,), jnp.int32),
            pltpu.VMEM((CB,), jnp.int32),
            pltpu.VMEM((CB, _DP), jnp.float32),
            pltpu.VMEM((CB, _DP), jnp.float32),
            pltpu.VMEM((_D, CB), jnp.float32),
            pltpu.VMEM((_D, CB), jnp.float32),
            pltpu.SemaphoreType.DMA,
            pltpu.SemaphoreType.DMA,
            pltpu.SemaphoreType.DMA,
            pltpu.SemaphoreType.DMA,
        ],
        compiler_params=pltpu.CompilerParams(
            use_tc_tiling_on_sc=True, needs_layout_passes=False
        ),
    )
    def k(tab_hbm, xt_hbm, out_hbm, idx_v, ig0, ig1, r0, r1,
          t0, t1, g0, g1, s0, s1):
        wid = lax.axis_index("s") * _NC + lax.axis_index("c")
        b0w = wid * b_per_w
        rows = (r0, r1)
        idxg = (ig0, ig1)
        tbuf = (t0, t1)
        gsem = (g0, g1)
        ssem = (s0, s1)
        lanes = lax.iota(jnp.int32, _L)
        rot = [(lanes + j) & (_L - 1) for j in range(_L)]

        for h in range(NH):
            pltpu.async_copy(
                xt_hbm.at[h, pl.ds(b0w, b_per_w)],
                idx_v.at[pl.ds(h * b_per_w, b_per_w)],
                g0,
            )
        for h in range(NH):
            pltpu.make_async_copy(
                xt_hbm.at[h, pl.ds(b0w, b_per_w)],
                idx_v.at[pl.ds(h * b_per_w, b_per_w)],
                g0,
            ).wait()
        def fill_idxg(b, off):
            for c in range(CB // _L):
                iv = idx_v[pl.ds(off + c * _L, _L)]
                idxg[b][pl.ds(c * _L, _L)] = lax.shift_right_logical(iv, 1)

        for b in range(2):
            fill_idxg(b, b * CB)
            pltpu.async_copy(tab_hbm.at[idxg[b]], rows[b], gsem[b])

        @pl.loop(0, ntask // 2)
        def _(i):
            for b in range(2):
                t = 2 * i + b

                @pl.when(t >= 2)
                def _():
                    pltpu.make_async_copy(
                        tbuf[b],
                        out_hbm.at[i - 1, :, pl.ds(b0w + b * CB, CB)],
                        ssem[b],
                    ).wait()

                pltpu.make_async_copy(
                    tab_hbm.at[pl.ds(0, CB)], rows[b], gsem[b]
                ).wait()

                toff = i * b_per_w + b * CB

                @pl.loop(0, CB // _L, unroll=4)
                def _(c):
                    bvec = jnp.full((_L,), c * _L, jnp.int32) + lanes
                    iv = idx_v[pl.ds(toff + c * _L, _L)]
                    par = lax.shift_left(iv & 1, 6)
                    for g in range(_D // _L):
                        for j in range(_L):
                            dvec = rot[j] + g * _L
                            v = plsc.load_gather(rows[b], [bvec, dvec + par])
                            plsc.store_scatter(tbuf[b], [dvec, bvec], v)

                pltpu.async_copy(
                    tbuf[b],
                    out_hbm.at[i, :, pl.ds(b0w + b * CB, CB)],
                    ssem[b],
                )

                @pl.when(t + 2 < ntask)
                def _():
                    fill_idxg(b, (i + 1) * b_per_w + b * CB)
                    pltpu.async_copy(tab_hbm.at[idxg[b]], rows[b], gsem[b])

        for b in range(2):
            pltpu.make_async_copy(
                tbuf[b],
                out_hbm.at[NH - 1, :, pl.ds(b0w + b * CB, CB)],
                ssem[b],
            ).wait()

    return k


def kernel(x, lut_weight):
    nb, nh = x.shape
    nv = lut_weight.shape[0]
    nfull = nv // _DP * _DP
    tl2 = lut_weight[nfull:].reshape((nv - nfull) // 2, _DP)
    tab = _format_kernel(nv, 384)(lut_weight.T, tl2)
    out3 = _gather_kernel(nb, nh, 256)(tab, x.T)
    return out3.transpose(2, 0, 1)
